# baseline (speedup 1.0000x reference)
import jax
import jax.numpy as jnp
from jax.experimental import pallas as pl
from jax.experimental.pallas import tpu as pltpu

BS, ROW, COL, D, H = 4, 512, 512, 128, 8
HD = D // H
MSH = 32
TANH_CLIP = 10.0
RB = 32
LOG2E = 1.4426950408889634


def _rne_bf16_f32(x):
    u = jax.lax.bitcast_convert_type(x, jnp.uint32)
    r = u + jnp.uint32(0x7FFF) + ((u >> 16) & jnp.uint32(1))
    return jax.lax.bitcast_convert_type(r & jnp.uint32(0xFFFF0000),
                                        jnp.float32)


def _body(row_ref, col_ref, cost_ref, wq_ref, bq_ref, wk_ref, bk_ref,
          wv_ref, bv_ref, wot_ref, bo_ref, beta_ref, w1_ref, b1_ref,
          w2_ref, b2_ref, out_ref, wts_ref):
    h = pl.program_id(1)
    rowb = row_ref[0]
    colb = col_ref[0]
    cost = cost_ref[0]
    dn_nt = (((1,), (1,)), ((), ()))
    dn_nn = (((1,), (0,)), ((), ()))
    f32 = jnp.float32
    hi = jax.lax.Precision.HIGHEST
    q = jax.lax.dot_general(rowb, wq_ref[...], dn_nt,
                            preferred_element_type=f32) + bq_ref[0]
    k = jax.lax.dot_general(colb, wk_ref[...], dn_nt,
                            preferred_element_type=f32) + bk_ref[0]
    v = jax.lax.dot_general(colb, wv_ref[...], dn_nt,
                            preferred_element_type=f32) + bv_ref[0]
    logits = jax.lax.dot_general(q * 0.25, k, dn_nt,
                                 preferred_element_type=f32, precision=hi)

    a = [w1_ref[0, 0, j] for j in range(MSH)]
    c = [w1_ref[0, 1, j] for j in range(MSH)]
    b1 = [b1_ref[0, 0, j] for j in range(MSH)]
    w2 = [w2_ref[0, 0, j] for j in range(MSH)]
    b2 = b2_ref[0, 0, 0]
    e_beta = jnp.exp2(beta_ref[0, 0, 0] * LOG2E)
    for i in range(ROW // RB):
        lg = logits[i * RB:(i + 1) * RB, :].astype(jnp.bfloat16).astype(f32)
        ctf = cost[i * RB:(i + 1) * RB, :]
        ct = ctf.astype(jnp.bfloat16).astype(f32)
        acc0 = jnp.full((RB, COL), b2, f32)
        acc1 = jnp.zeros((RB, COL), f32)
        for j in range(0, MSH, 2):
            h0 = jnp.maximum((lg * a[j] + ct * c[j]) + b1[j], 0.0)
            h1 = jnp.maximum((lg * a[j + 1] + ct * c[j + 1]) + b1[j + 1], 0.0)
            acc0 = acc0 + h0.astype(jnp.bfloat16).astype(f32) * w2[j]
            acc1 = acc1 + h1.astype(jnp.bfloat16).astype(f32) * w2[j + 1]
        t = jnp.tanh(acc0 + acc1)
        e_i = jnp.where(ctf > 0.0, jnp.exp2(t * (TANH_CLIP * LOG2E)), e_beta)
        s_i = jnp.sum(e_i, axis=1, keepdims=True)
        wts_ref[i * RB:(i + 1) * RB, :] = (
            e_i * (1.0 / s_i)).astype(jnp.bfloat16)
    head = jax.lax.dot_general(wts_ref[...], v.astype(jnp.bfloat16), dn_nn,
                               preferred_element_type=f32)
    contrib = jax.lax.dot_general(head.astype(jnp.bfloat16), wot_ref[...],
                                  dn_nn, preferred_element_type=f32)

    @pl.when(h == 0)
    def _init():
        out_ref[0] = contrib + bo_ref[0]

    @pl.when(h != 0)
    def _accum():
        out_ref[0] = out_ref[0] + contrib


@jax.jit
def kernel(row_emb, col_emb, cost_mat, W_q, b_q, W_k, b_k, W_v, b_v,
           W_o, b_o, beta, ms_W1, ms_b1, ms_W2, ms_b2):
    grid = (BS, H)
    bf16 = jnp.bfloat16
    w_spec = pl.BlockSpec((HD, D), lambda b, h: (h, 0))
    bias_spec = pl.BlockSpec((1, 1, HD), lambda b, h: (h, 0, 0))
    hs_spec = pl.BlockSpec((1, 1, MSH), lambda b, h: (h, 0, 0),
                           memory_space=pltpu.SMEM)
    scal_spec = pl.BlockSpec((1, 1, 1), lambda b, h: (h, 0, 0),
                             memory_space=pltpu.SMEM)
    out = pl.pallas_call(
        _body,
        grid=grid,
        in_specs=[
            pl.BlockSpec((1, ROW, D), lambda b, h: (b, 0, 0)),
            pl.BlockSpec((1, COL, D), lambda b, h: (b, 0, 0)),
            pl.BlockSpec((1, ROW, COL), lambda b, h: (b, 0, 0)),
            w_spec,
            bias_spec,
            w_spec,
            bias_spec,
            w_spec,
            bias_spec,
            w_spec,
            pl.BlockSpec((1, D), lambda b, h: (0, 0)),
            scal_spec,
            pl.BlockSpec((1, 2, MSH), lambda b, h: (h, 0, 0),
                         memory_space=pltpu.SMEM),
            hs_spec,
            hs_spec,
            scal_spec,
        ],
        out_specs=pl.BlockSpec((1, ROW, D), lambda b, h: (b, 0, 0)),
        out_shape=jax.ShapeDtypeStruct((BS, ROW, D), jnp.float32),
        scratch_shapes=[pltpu.VMEM((ROW, COL), bf16)],
    )(
        row_emb.astype(bf16), col_emb.astype(bf16), cost_mat,
        W_q.astype(bf16), b_q.reshape(H, 1, HD),
        W_k.astype(bf16), b_k.reshape(H, 1, HD),
        W_v.astype(bf16), b_v.reshape(H, 1, HD),
        W_o.T.astype(bf16), b_o.reshape(1, D),
        beta.reshape(H, 1, 1),
        _rne_bf16_f32(ms_W1), ms_b1.reshape(H, 1, MSH),
        _rne_bf16_f32(ms_W2.reshape(H, 1, MSH)), ms_b2.reshape(H, 1, 1),
    )
    return out


# speedup vs baseline: 1.0062x; 1.0062x over previous
import jax
import jax.numpy as jnp
from jax.experimental import pallas as pl
from jax.experimental.pallas import tpu as pltpu

BS, ROW, COL, D, H = 4, 512, 512, 128, 8
HD = D // H
MSH = 32
TANH_CLIP = 10.0
RB = 32
LOG2E = 1.4426950408889634


def _rne_bf16_f32(x):
    u = jax.lax.bitcast_convert_type(x, jnp.uint32)
    r = u + jnp.uint32(0x7FFF) + ((u >> 16) & jnp.uint32(1))
    return jax.lax.bitcast_convert_type(r & jnp.uint32(0xFFFF0000),
                                        jnp.float32)


def _body(row_ref, col_ref, cost_ref, wq_ref, bq_ref, wk_ref, bk_ref,
          wv_ref, bv_ref, wot_ref, bo_ref, beta_ref, w1_ref, b1_ref,
          w2_ref, b2_ref, out_ref, wts_ref):
    h = pl.program_id(1)
    rowb = row_ref[0]
    colb = col_ref[0]
    cost = cost_ref[0]
    dn_nt = (((1,), (1,)), ((), ()))
    dn_nn = (((1,), (0,)), ((), ()))
    f32 = jnp.float32
    hi = jax.lax.Precision.HIGHEST
    q = jax.lax.dot_general(rowb, wq_ref[...], dn_nt,
                            preferred_element_type=f32) + bq_ref[0]
    k = jax.lax.dot_general(colb, wk_ref[...], dn_nt,
                            preferred_element_type=f32) + bk_ref[0]
    v = jax.lax.dot_general(colb, wv_ref[...], dn_nt,
                            preferred_element_type=f32) + bv_ref[0]
    logits = jax.lax.dot_general(q * 0.25, k, dn_nt,
                                 preferred_element_type=f32, precision=hi)

    a = [w1_ref[0, 0, j] for j in range(MSH)]
    c = [w1_ref[0, 1, j] for j in range(MSH)]
    b1 = [b1_ref[0, 0, j] for j in range(MSH)]
    w2 = [w2_ref[0, 0, j] for j in range(MSH)]
    b2 = b2_ref[0, 0, 0]
    e_beta = jnp.exp2(beta_ref[0, 0, 0] * LOG2E)
    for i in range(ROW // RB):
        lg = logits[i * RB:(i + 1) * RB, :].astype(jnp.bfloat16).astype(f32)
        ctf = cost[i * RB:(i + 1) * RB, :]
        ct = ctf.astype(jnp.bfloat16).astype(f32)
        acc = [jnp.full((RB, COL), b2, f32), jnp.zeros((RB, COL), f32),
               jnp.zeros((RB, COL), f32), jnp.zeros((RB, COL), f32)]
        for j in range(0, MSH, 4):
            for u in range(4):
                hj = jnp.maximum((lg * a[j + u] + ct * c[j + u]) + b1[j + u],
                                 0.0)
                acc[u] = acc[u] + hj.astype(jnp.bfloat16).astype(f32) * w2[j + u]
        t = jnp.tanh((acc[0] + acc[1]) + (acc[2] + acc[3]))
        e_i = jnp.where(ctf > 0.0, jnp.exp2(t * (TANH_CLIP * LOG2E)), e_beta)
        s_i = jnp.sum(e_i, axis=1, keepdims=True)
        wts_ref[i * RB:(i + 1) * RB, :] = (
            e_i * (1.0 / s_i)).astype(jnp.bfloat16)
    head = jax.lax.dot_general(wts_ref[...], v.astype(jnp.bfloat16), dn_nn,
                               preferred_element_type=f32)
    contrib = jax.lax.dot_general(head.astype(jnp.bfloat16), wot_ref[...],
                                  dn_nn, preferred_element_type=f32)

    @pl.when(h == 0)
    def _init():
        out_ref[0] = contrib + bo_ref[0]

    @pl.when(h != 0)
    def _accum():
        out_ref[0] = out_ref[0] + contrib


@jax.jit
def kernel(row_emb, col_emb, cost_mat, W_q, b_q, W_k, b_k, W_v, b_v,
           W_o, b_o, beta, ms_W1, ms_b1, ms_W2, ms_b2):
    grid = (BS, H)
    bf16 = jnp.bfloat16
    w_spec = pl.BlockSpec((HD, D), lambda b, h: (h, 0))
    bias_spec = pl.BlockSpec((1, 1, HD), lambda b, h: (h, 0, 0))
    hs_spec = pl.BlockSpec((1, 1, MSH), lambda b, h: (h, 0, 0),
                           memory_space=pltpu.SMEM)
    scal_spec = pl.BlockSpec((1, 1, 1), lambda b, h: (h, 0, 0),
                             memory_space=pltpu.SMEM)
    out = pl.pallas_call(
        _body,
        grid=grid,
        in_specs=[
            pl.BlockSpec((1, ROW, D), lambda b, h: (b, 0, 0)),
            pl.BlockSpec((1, COL, D), lambda b, h: (b, 0, 0)),
            pl.BlockSpec((1, ROW, COL), lambda b, h: (b, 0, 0)),
            w_spec,
            bias_spec,
            w_spec,
            bias_spec,
            w_spec,
            bias_spec,
            w_spec,
            pl.BlockSpec((1, D), lambda b, h: (0, 0)),
            scal_spec,
            pl.BlockSpec((1, 2, MSH), lambda b, h: (h, 0, 0),
                         memory_space=pltpu.SMEM),
            hs_spec,
            hs_spec,
            scal_spec,
        ],
        out_specs=pl.BlockSpec((1, ROW, D), lambda b, h: (b, 0, 0)),
        out_shape=jax.ShapeDtypeStruct((BS, ROW, D), jnp.float32),
        scratch_shapes=[pltpu.VMEM((ROW, COL), bf16)],
    )(
        row_emb.astype(bf16), col_emb.astype(bf16), cost_mat,
        W_q.astype(bf16), b_q.reshape(H, 1, HD),
        W_k.astype(bf16), b_k.reshape(H, 1, HD),
        W_v.astype(bf16), b_v.reshape(H, 1, HD),
        W_o.T.astype(bf16), b_o.reshape(1, D),
        beta.reshape(H, 1, 1),
        _rne_bf16_f32(ms_W1), ms_b1.reshape(H, 1, MSH),
        _rne_bf16_f32(ms_W2.reshape(H, 1, MSH)), ms_b2.reshape(H, 1, 1),
    )
    return out
